# baseline (speedup 1.0000x reference)
import functools
import jax
import jax.numpy as jnp
from jax import lax
from jax.experimental import pallas as pl
from jax.experimental.pallas import tpu as pltpu
from jax.experimental.pallas import tpu_sc as plsc

N_NODES = 10000
N_ROWS = 10016
PAD_DST = 10008
N_EDGES = 320000
D_IN = 128
D_HID = 128
N_CLASS = 40
D_PAD = 48

N_SC = 2
N_TILES = 16
N_WORKERS = N_SC * N_TILES
ROWS_PER_TILE = N_ROWS // N_TILES
ROW_BLK = 2504


def _edge_aggregate(sup, src2d, dst2d, d, chunk, slots, nbuf):
    mesh = plsc.VectorSubcoreMesh(core_axis_name="c", subcore_axis_name="s",
                                  num_cores=N_SC, num_subcores=N_TILES)

    def body(sup_hbm, src_hbm, dst_hbm, zeros_hbm, out_hbm,
             sidx_v, didx_v, rows_v, acc_sh, sem_g, sem_s):
        c = lax.axis_index("c")
        s = lax.axis_index("s")
        wid = c * N_TILES + s
        start = wid * slots

        pltpu.sync_copy(src_hbm.at[pl.ds(start, slots)], sidx_v)
        pltpu.sync_copy(dst_hbm.at[pl.ds(start, slots)], didx_v)
        pltpu.sync_copy(zeros_hbm,
                        acc_sh.at[pl.ds(s * ROWS_PER_TILE, ROWS_PER_TILE)])
        plsc.subcore_barrier()

        def fire_gather(t, b):
            pltpu.async_copy(sup_hbm.at[sidx_v.at[t]], rows_v.at[b],
                             sem_g.at[b])

        def wait_gather(t, b):
            pltpu.make_async_copy(sup_hbm.at[sidx_v.at[t]], rows_v.at[b],
                                  sem_g.at[b]).wait()

        def fire_scatter(t, b):
            pltpu.async_copy(rows_v.at[b], acc_sh.at[didx_v.at[t]],
                             sem_s.at[b], add=True)

        def wait_scatter(t, b):
            pltpu.make_async_copy(rows_v.at[b], acc_sh.at[didx_v.at[t]],
                                  sem_s.at[b]).wait()

        for b in range(nbuf - 1):
            fire_gather(b, b)

        def slot_step(t, sb, first=False, last_grp=False):
            sbp = (sb + nbuf - 1) % nbuf
            wait_gather(t, sb)
            fire_scatter(t, sb)
            if not first:
                wait_scatter(t - 1, sbp)
            if not last_grp:
                fire_gather(t + nbuf - 1, sbp)

        for sb in range(nbuf):
            slot_step(sb, sb, first=(sb == 0))

        def group(g, carry):
            t0 = g * nbuf
            for sb in range(nbuf):
                slot_step(t0 + sb, sb)
            return carry

        lax.fori_loop(1, slots // nbuf - 1, group, 0)

        t0 = slots - nbuf
        for sb in range(nbuf):
            slot_step(t0 + sb, sb, last_grp=(sb != 0))
        wait_scatter(slots - 1, (slots - 1) % nbuf)

        plsc.subcore_barrier()
        pltpu.sync_copy(acc_sh.at[pl.ds(s * ROWS_PER_TILE, ROWS_PER_TILE)],
                        out_hbm.at[c, s])

    kern = pl.kernel(
        body,
        out_type=jax.ShapeDtypeStruct((N_SC, N_TILES, ROWS_PER_TILE, d),
                                      jnp.float32),
        mesh=mesh,
        scratch_types=[
            pltpu.VMEM((slots, chunk), jnp.int32),
            pltpu.VMEM((slots, chunk), jnp.int32),
            pltpu.VMEM((nbuf, chunk, d), jnp.float32),
            pltpu.VMEM_SHARED((N_ROWS, d), jnp.float32),
            pltpu.SemaphoreType.DMA((nbuf,)),
            pltpu.SemaphoreType.DMA((nbuf,)),
        ],
        compiler_params=pltpu.CompilerParams(use_tc_tiling_on_sc=False),
    )
    zeros = jnp.zeros((ROWS_PER_TILE, d), jnp.float32)
    out = kern(sup, src2d, dst2d, zeros)
    return out.reshape(N_SC, N_ROWS, d)


def _pad_edges(src, dst, chunk, slots):
    n_pad = N_WORKERS * slots * chunk - N_EDGES
    src2d = jnp.concatenate(
        [src, jnp.zeros((n_pad,), jnp.int32)]).reshape(-1, chunk)
    dst2d = jnp.concatenate(
        [dst, jnp.full((n_pad,), PAD_DST, jnp.int32)]).reshape(-1, chunk)
    return src2d, dst2d


def _matmul1(x, w):
    def body(x_ref, w_ref, o_ref):
        o_ref[...] = jnp.dot(x_ref[...], w_ref[...],
                             preferred_element_type=jnp.float32)

    return pl.pallas_call(
        body,
        grid=(4,),
        in_specs=[
            pl.BlockSpec((ROW_BLK, D_IN), lambda i: (i, 0)),
            pl.BlockSpec((D_IN, D_HID), lambda i: (0, 0)),
        ],
        out_specs=pl.BlockSpec((ROW_BLK, D_HID), lambda i: (i, 0)),
        out_shape=jax.ShapeDtypeStruct((N_ROWS, D_HID), jnp.float32),
    )(x, w)


def _mid(p, b1, w2p):
    def body(p_ref, b_ref, w_ref, o_ref):
        h = jnp.maximum(p_ref[0] + p_ref[1] + b_ref[...], 0.0)
        o_ref[...] = jnp.dot(h, w_ref[...], preferred_element_type=jnp.float32)

    return pl.pallas_call(
        body,
        grid=(4,),
        in_specs=[
            pl.BlockSpec((N_SC, ROW_BLK, D_HID), lambda i: (0, i, 0)),
            pl.BlockSpec((1, D_HID), lambda i: (0, 0)),
            pl.BlockSpec((D_HID, D_PAD), lambda i: (0, 0)),
        ],
        out_specs=pl.BlockSpec((ROW_BLK, D_PAD), lambda i: (i, 0)),
        out_shape=jax.ShapeDtypeStruct((N_ROWS, D_PAD), jnp.float32),
    )(p, b1, w2p)


def _final(q, b2p):
    def body(q_ref, b_ref, o_ref):
        o_ref[...] = q_ref[0] + q_ref[1] + b_ref[...]

    return pl.pallas_call(
        body,
        grid=(4,),
        in_specs=[
            pl.BlockSpec((N_SC, ROW_BLK, D_PAD), lambda i: (0, i, 0)),
            pl.BlockSpec((1, D_PAD), lambda i: (0, 0)),
        ],
        out_specs=pl.BlockSpec((ROW_BLK, D_PAD), lambda i: (i, 0)),
        out_shape=jax.ShapeDtypeStruct((N_ROWS, D_PAD), jnp.float32),
    )(q, b2p)


@jax.jit
def kernel(x, adj, W1, b1, W2, b2):
    src1, dst1 = _pad_edges(adj[0], adj[1], 112, 90)
    src2, dst2 = _pad_edges(adj[0], adj[1], 128, 80)
    xp = jnp.pad(x, ((0, N_ROWS - N_NODES), (0, 0)))
    w2p = jnp.pad(W2, ((0, 0), (0, D_PAD - N_CLASS)))
    b2p = jnp.pad(b2, (0, D_PAD - N_CLASS)).reshape(1, D_PAD)
    b1r = b1.reshape(1, D_HID)

    support1 = _matmul1(xp, W1)
    p1 = _edge_aggregate(support1, src1, dst1, D_HID, 112, 90, 2)
    support2 = _mid(p1, b1r, w2p)
    q = _edge_aggregate(support2, src2, dst2, D_PAD, 128, 80, 4)
    out = _final(q, b2p)
    return out[:N_NODES, :N_CLASS]
